# baseline (speedup 1.0000x reference)
import functools

import jax
import jax.numpy as jnp
from jax import lax
from jax.experimental import pallas as pl
from jax.experimental.pallas import tpu as pltpu
from jax.experimental.pallas import tpu_sc as plsc

_N_NODES = 10000
_N_EDGES = 320000
_NODE_DIM = 256
_EDGE_DIM = 128
_N_POS = 128
_N_ATTR = 64
_N_COMB = _N_POS * _N_ATTR
_NODE_PAD = 1024

_NC, _NS = 2, 16
_NW = _NC * _NS

_BE = _N_EDGES // _NW
_CE = 80
_NBUF = 5
_NCH = _BE // _CE

_XBLK = 400
_XG = _N_NODES // _XBLK


def _x_body(x_ref, nt_ref, out_ref):
    xb = x_ref[0, 0, :]
    onehot = (xb[:, None] == lax.broadcasted_iota(jnp.int32, (_XBLK, _NODE_PAD), 1))
    out_ref[:] = jnp.dot(onehot.astype(jnp.float32), nt_ref[:],
                         preferred_element_type=jnp.float32)


_x_emb_call = pl.pallas_call(
    _x_body,
    grid=(_XG,),
    in_specs=[
        pl.BlockSpec((1, 1, _XBLK), lambda i: (i, 0, 0)),
        pl.BlockSpec((_NODE_PAD, _NODE_DIM), lambda i: (0, 0)),
    ],
    out_specs=pl.BlockSpec((_XBLK, _NODE_DIM), lambda i: (i, 0)),
    out_shape=jax.ShapeDtypeStruct((_N_NODES, _NODE_DIM), jnp.float32),
)


_mesh = plsc.VectorSubcoreMesh(core_axis_name="c", subcore_axis_name="s")


@functools.partial(
    pl.kernel,
    out_type=jax.ShapeDtypeStruct((_N_EDGES, _EDGE_DIM), jnp.float32),
    mesh=_mesh,
    scratch_types=(
        [pltpu.VMEM((_BE,), jnp.int32)]
        + [pltpu.VMEM((_CE, _EDGE_DIM), jnp.float32) for _ in range(_NBUF)]
        + [pltpu.SemaphoreType.DMA for _ in range(2 * _NBUF)]
        + [pltpu.VMEM_SHARED((_N_COMB, _EDGE_DIM), jnp.float32)]
    ),
)
def _sc_gather(e_hbm, pos_hbm, edge_hbm, e_out, eidx_all, *rest):
    bufs = list(rest[:_NBUF])
    isems = list(rest[_NBUF:2 * _NBUF])
    osems = list(rest[2 * _NBUF:3 * _NBUF])
    comb_sh = rest[3 * _NBUF]
    wid = lax.axis_index("s") * _NC + lax.axis_index("c")
    sid = lax.axis_index("s")

    ebase = pl.multiple_of(wid * _BE, 8)
    eload = pltpu.async_copy(e_hbm.at[pl.ds(ebase, _BE)], eidx_all, isems[0])
    pltpu.sync_copy(edge_hbm, bufs[1].at[pl.ds(0, _N_ATTR)])
    pb = pl.multiple_of(sid * 8, 8)
    pltpu.sync_copy(pos_hbm.at[pl.ds(pb, 8)], bufs[2].at[pl.ds(0, 8)])
    for p in range(8):
        pvs = [bufs[2][p, pl.ds(c * 16, 16)] for c in range(8)]

        def arow(a, carry, pvs=pvs):
            for c in range(8):
                s = pl.ds(c * 16, 16)
                bufs[0][a, s] = pvs[c] + bufs[1][a, s]
            return carry

        lax.fori_loop(0, _N_ATTR, arow, 0)
        rb = pl.multiple_of((sid * 8 + p) * _N_ATTR, 8)
        pltpu.sync_copy(bufs[0].at[pl.ds(0, _N_ATTR)],
                        comb_sh.at[pl.ds(rb, _N_ATTR)])
    eload.wait()
    plsc.subcore_barrier()

    def gather_in(i, rows_v, sem):
        off = pl.multiple_of(i * _CE, 8)
        return pltpu.async_copy(comb_sh.at[eidx_all.at[pl.ds(off, _CE)]],
                                rows_v, sem)

    def copy_out(i, rows_v, sem):
        b = pl.multiple_of(ebase + i * _CE, 8)
        pltpu.async_copy(rows_v, e_out.at[pl.ds(b, _CE)], sem)

    def wait_out(rows_v, sem):
        pltpu.make_async_copy(rows_v, e_out.at[pl.ds(0, _CE)], sem).wait()

    hs = [gather_in(k, bufs[k], isems[k]) for k in range(_NBUF)]
    for k in range(_NBUF):
        hs[k].wait()
        copy_out(k, bufs[k], osems[k])

    def ring_body(g, carry):
        c0 = g * _NBUF
        hh = []
        for k in range(_NBUF):
            wait_out(bufs[k], osems[k])
            hh.append(gather_in(c0 + k, bufs[k], isems[k]))
        for k in range(_NBUF):
            hh[k].wait()
            copy_out(c0 + k, bufs[k], osems[k])
        return carry

    lax.fori_loop(1, _NCH // _NBUF, ring_body, 0)

    for k in range(_NBUF):
        wait_out(bufs[k], osems[k])


def kernel(x, e, node_table, edge_table, pos_table):
    e_emb = _sc_gather(e.astype(jnp.int32), pos_table, edge_table)
    node_p = jnp.pad(node_table, ((0, _NODE_PAD - node_table.shape[0]), (0, 0)))
    x3 = x.astype(jnp.int32).reshape(_XG, 1, _XBLK)
    x_emb = _x_emb_call(x3, node_p)
    return (x_emb, e_emb)


# speedup vs baseline: 1.0366x; 1.0366x over previous
import functools

import jax
import jax.numpy as jnp
from jax import lax
from jax.experimental import pallas as pl
from jax.experimental.pallas import tpu as pltpu
from jax.experimental.pallas import tpu_sc as plsc

_N_NODES = 10000
_N_EDGES = 320000
_NODE_DIM = 256
_EDGE_DIM = 128
_N_POS = 128
_N_ATTR = 64
_N_COMB = _N_POS * _N_ATTR
_NODE_PAD = 1000

_NC, _NS = 2, 16
_NW = _NC * _NS

_BE = _N_EDGES // _NW
_CE = 80
_NBUF = 5
_NCH = _BE // _CE

_XBLK = 400
_XG = _N_NODES // _XBLK


def _combine_body(pos_ref, edge_ref, out_ref):
    out_ref[:] = pos_ref[:][:, None, :] + edge_ref[:][None, :, :]


def _build_combined(pos_table, edge_table):
    out3 = pl.pallas_call(
        _combine_body,
        out_shape=jax.ShapeDtypeStruct((_N_POS, _N_ATTR, _EDGE_DIM), jnp.float32),
    )(pos_table, edge_table)
    return out3.reshape(_N_COMB, _EDGE_DIM)


def _x_body(x_ref, nt_ref, out_ref):
    xb = x_ref[0, 0, :]
    onehot = (xb[:, None] == lax.broadcasted_iota(jnp.int32, (_XBLK, _NODE_PAD), 1))
    out_ref[:] = jnp.dot(onehot.astype(jnp.float32), nt_ref[:],
                         preferred_element_type=jnp.float32)


_x_emb_call = pl.pallas_call(
    _x_body,
    grid=(_XG,),
    in_specs=[
        pl.BlockSpec((1, 1, _XBLK), lambda i: (i, 0, 0)),
        pl.BlockSpec((_NODE_PAD, _NODE_DIM), lambda i: (0, 0)),
    ],
    out_specs=pl.BlockSpec((_XBLK, _NODE_DIM), lambda i: (i, 0)),
    out_shape=jax.ShapeDtypeStruct((_N_NODES, _NODE_DIM), jnp.float32),
)


_mesh = plsc.VectorSubcoreMesh(core_axis_name="c", subcore_axis_name="s")


@functools.partial(
    pl.kernel,
    out_type=jax.ShapeDtypeStruct((_N_EDGES, _EDGE_DIM), jnp.float32),
    mesh=_mesh,
    scratch_types=(
        [pltpu.VMEM((_BE,), jnp.int32)]
        + [pltpu.VMEM((_CE, _EDGE_DIM), jnp.float32) for _ in range(_NBUF)]
        + [pltpu.SemaphoreType.DMA for _ in range(2 * _NBUF)]
        + [pltpu.VMEM_SHARED((_N_COMB, _EDGE_DIM), jnp.float32)]
    ),
)
def _sc_gather(e_hbm, comb_hbm, e_out, eidx_all, *rest):
    bufs = list(rest[:_NBUF])
    isems = list(rest[_NBUF:2 * _NBUF])
    osems = list(rest[2 * _NBUF:3 * _NBUF])
    comb_sh = rest[3 * _NBUF]
    wid = lax.axis_index("s") * _NC + lax.axis_index("c")
    sid = lax.axis_index("s")

    cb = pl.multiple_of(sid * (_N_COMB // _NS), 8)
    stage = pltpu.async_copy(comb_hbm.at[pl.ds(cb, _N_COMB // _NS)],
                             comb_sh.at[pl.ds(cb, _N_COMB // _NS)], isems[0])
    ebase = pl.multiple_of(wid * _BE, 8)
    pltpu.sync_copy(e_hbm.at[pl.ds(ebase, _BE)], eidx_all)
    stage.wait()
    plsc.subcore_barrier()

    def gather_in(i, rows_v, sem):
        off = pl.multiple_of(i * _CE, 8)
        return pltpu.async_copy(comb_sh.at[eidx_all.at[pl.ds(off, _CE)]],
                                rows_v, sem)

    def copy_out(i, rows_v, sem):
        b = pl.multiple_of(ebase + i * _CE, 8)
        pltpu.async_copy(rows_v, e_out.at[pl.ds(b, _CE)], sem)

    def wait_out(rows_v, sem):
        pltpu.make_async_copy(rows_v, e_out.at[pl.ds(0, _CE)], sem).wait()

    hs = [gather_in(k, bufs[k], isems[k]) for k in range(_NBUF)]
    for k in range(_NBUF):
        hs[k].wait()
        copy_out(k, bufs[k], osems[k])

    def ring_body(g, carry):
        c0 = g * _NBUF
        hh = []
        for k in range(_NBUF):
            wait_out(bufs[k], osems[k])
            hh.append(gather_in(c0 + k, bufs[k], isems[k]))
        for k in range(_NBUF):
            hh[k].wait()
            copy_out(c0 + k, bufs[k], osems[k])
        return carry

    lax.fori_loop(1, _NCH // _NBUF, ring_body, 0)

    for k in range(_NBUF):
        wait_out(bufs[k], osems[k])


def kernel(x, e, node_table, edge_table, pos_table):
    combined = _build_combined(pos_table, edge_table)
    e_emb = _sc_gather(e.astype(jnp.int32), combined)
    x3 = x.astype(jnp.int32).reshape(_XG, 1, _XBLK)
    x_emb = _x_emb_call(x3, node_table)
    return (x_emb, e_emb)
